# baseline (speedup 1.0000x reference)
import jax
import jax.numpy as jnp
from jax.experimental import pallas as pl
from jax.experimental.pallas import tpu as pltpu

N_TILE = 2304


def _mq_kernel(q_ref, w_ref, wt_ref, emb_ref, idx_ref, oh_ref, perp_ref,
               idx_s, counts_ref):
    i = pl.program_id(0)
    nsteps = pl.num_programs(0) - 1
    K = w_ref.shape[0]

    @pl.when(i > 0)
    def _():
        idx_prev = idx_s[:]
        iota = jax.lax.broadcasted_iota(jnp.int32, (idx_prev.shape[0], K), 1)
        oh = (iota == idx_prev).astype(jnp.float32)
        oh_ref[:] = oh
        emb_ref[:] = jnp.dot(oh, w_ref[:], preferred_element_type=jnp.float32)
        tile_counts = jnp.sum(oh, axis=0, keepdims=True)

        @pl.when(i == 1)
        def _():
            counts_ref[:] = tile_counts

        @pl.when(i > 1)
        def _():
            counts_ref[:] = counts_ref[:] + tile_counts

    @pl.when(i < nsteps)
    def _():
        q = q_ref[:]
        wt = wt_ref[:]
        sim = jnp.dot(q, wt, preferred_element_type=jnp.float32)
        l2q = jnp.sum(q * q, axis=1, keepdims=True)
        l2k = jnp.sum(wt * wt, axis=0, keepdims=True)
        dist = (l2q + l2k) - 2.0 * sim
        idx = jnp.argmin(dist, axis=1).astype(jnp.int32)
        idx_ref[:] = idx[:, None]
        idx_s[:] = idx[:, None]

    @pl.when(i == nsteps)
    def _():
        n_total = nsteps * q_ref.shape[0]
        z_mean = counts_ref[:] / n_total
        ent = jnp.sum(z_mean * jnp.log(z_mean + 1e-10), axis=1, keepdims=True)
        perp_ref[:] = jnp.exp(-ent)


def kernel(x, W):
    b, n, c = x.shape
    q = x.reshape(b * n, c)
    N = b * n
    K = W.shape[0]
    wt = W.T
    nsteps = N // N_TILE

    def cur(i):
        return jnp.minimum(i, nsteps - 1)

    def prev(i):
        return jnp.maximum(i - 1, 0)

    emb, idx, oh, perp = pl.pallas_call(
        _mq_kernel,
        grid=(nsteps + 1,),
        in_specs=[
            pl.BlockSpec((N_TILE, c), lambda i: (cur(i), 0)),
            pl.BlockSpec((K, c), lambda i: (0, 0)),
            pl.BlockSpec((c, K), lambda i: (0, 0)),
        ],
        out_specs=[
            pl.BlockSpec((N_TILE, c), lambda i: (prev(i), 0)),
            pl.BlockSpec((N_TILE, 1), lambda i: (cur(i), 0)),
            pl.BlockSpec((N_TILE, K), lambda i: (prev(i), 0)),
            pl.BlockSpec((1, 1), lambda i: (0, 0)),
        ],
        out_shape=[
            jax.ShapeDtypeStruct((N, c), jnp.float32),
            jax.ShapeDtypeStruct((N, 1), jnp.int32),
            jax.ShapeDtypeStruct((N, K), jnp.float32),
            jax.ShapeDtypeStruct((1, 1), jnp.float32),
        ],
        scratch_shapes=[
            pltpu.VMEM((N_TILE, 1), jnp.int32),
            pltpu.VMEM((1, K), jnp.float32),
        ],
    )(q, W, wt)

    embedding = emb.reshape(b, n, c)
    indices = idx.reshape(b, n)
    onehot = oh.reshape(b, n, K)
    perplexity = perp[0, 0]
    return embedding, indices, onehot, perplexity


# speedup vs baseline: 1.0177x; 1.0177x over previous
import jax
import jax.numpy as jnp
from jax.experimental import pallas as pl
from jax.experimental.pallas import tpu as pltpu

N_TILE = 1024


def _mq_kernel(q_ref, w_ref, wt_ref, emb_ref, idx_ref, oh_ref, perp_ref,
               idx_s, counts_ref):
    i = pl.program_id(0)
    nsteps = pl.num_programs(0) - 1
    K = w_ref.shape[0]

    @pl.when(i > 0)
    def _():
        idx_prev = idx_s[:]
        iota = jax.lax.broadcasted_iota(jnp.int32, (idx_prev.shape[0], K), 1)
        oh = (iota == idx_prev).astype(jnp.float32)
        oh_ref[:] = oh
        emb_ref[:] = jnp.dot(oh, w_ref[:], preferred_element_type=jnp.float32)
        tile_counts = jnp.sum(oh, axis=0, keepdims=True)

        @pl.when(i == 1)
        def _():
            counts_ref[:] = tile_counts

        @pl.when(i > 1)
        def _():
            counts_ref[:] = counts_ref[:] + tile_counts

    @pl.when(i < nsteps)
    def _():
        q = q_ref[:]
        wt = wt_ref[:]
        sim = jnp.dot(q, wt, preferred_element_type=jnp.float32)
        l2q = jnp.sum(q * q, axis=1, keepdims=True)
        l2k = jnp.sum(wt * wt, axis=0, keepdims=True)
        dist = (l2q + l2k) - 2.0 * sim
        idx = jnp.argmin(dist, axis=1).astype(jnp.int32)
        idx_ref[:] = idx[:, None]
        idx_s[:] = idx[:, None]

    @pl.when(i == nsteps)
    def _():
        n_total = nsteps * q_ref.shape[0]
        z_mean = counts_ref[:] / n_total
        ent = jnp.sum(z_mean * jnp.log(z_mean + 1e-10), axis=1, keepdims=True)
        perp_ref[:] = jnp.exp(-ent)


def kernel(x, W):
    b, n, c = x.shape
    q = x.reshape(b * n, c)
    N = b * n
    K = W.shape[0]
    wt = W.T
    nsteps = N // N_TILE

    def cur(i):
        return jnp.minimum(i, nsteps - 1)

    def prev(i):
        return jnp.maximum(i - 1, 0)

    emb, idx, oh, perp = pl.pallas_call(
        _mq_kernel,
        grid=(nsteps + 1,),
        in_specs=[
            pl.BlockSpec((N_TILE, c), lambda i: (cur(i), 0)),
            pl.BlockSpec((K, c), lambda i: (0, 0)),
            pl.BlockSpec((c, K), lambda i: (0, 0)),
        ],
        out_specs=[
            pl.BlockSpec((N_TILE, c), lambda i: (prev(i), 0)),
            pl.BlockSpec((N_TILE, 1), lambda i: (cur(i), 0)),
            pl.BlockSpec((N_TILE, K), lambda i: (prev(i), 0)),
            pl.BlockSpec((1, 1), lambda i: (0, 0)),
        ],
        out_shape=[
            jax.ShapeDtypeStruct((N, c), jnp.float32),
            jax.ShapeDtypeStruct((N, 1), jnp.int32),
            jax.ShapeDtypeStruct((N, K), jnp.float32),
            jax.ShapeDtypeStruct((1, 1), jnp.float32),
        ],
        scratch_shapes=[
            pltpu.VMEM((N_TILE, 1), jnp.int32),
            pltpu.VMEM((1, K), jnp.float32),
        ],
    )(q, W, wt)

    embedding = emb.reshape(b, n, c)
    indices = idx.reshape(b, n)
    onehot = oh.reshape(b, n, K)
    perplexity = perp[0, 0]
    return embedding, indices, onehot, perplexity


# speedup vs baseline: 1.0286x; 1.0107x over previous
import jax
import jax.numpy as jnp
from jax.experimental import pallas as pl
from jax.experimental.pallas import tpu as pltpu

N_TILE = 1536


def _mq_kernel(q_ref, w_ref, wt_ref, emb_ref, idx_ref, oh_ref, perp_ref,
               idx_s, counts_ref):
    i = pl.program_id(0)
    nsteps = pl.num_programs(0) - 1
    K = w_ref.shape[0]

    @pl.when(i > 0)
    def _():
        idx_prev = idx_s[:]
        iota = jax.lax.broadcasted_iota(jnp.int32, (idx_prev.shape[0], K), 1)
        oh = (iota == idx_prev).astype(jnp.float32)
        oh_ref[:] = oh
        emb_ref[:] = jnp.dot(oh, w_ref[:], preferred_element_type=jnp.float32)
        tile_counts = jnp.sum(oh, axis=0, keepdims=True)

        @pl.when(i == 1)
        def _():
            counts_ref[:] = tile_counts

        @pl.when(i > 1)
        def _():
            counts_ref[:] = counts_ref[:] + tile_counts

    @pl.when(i < nsteps)
    def _():
        q = q_ref[:]
        wt = wt_ref[:]
        sim = jnp.dot(q, wt, preferred_element_type=jnp.float32)
        l2q = jnp.sum(q * q, axis=1, keepdims=True)
        l2k = jnp.sum(wt * wt, axis=0, keepdims=True)
        dist = (l2q + l2k) - 2.0 * sim
        idx = jnp.argmin(dist, axis=1).astype(jnp.int32)
        idx_ref[:] = idx[:, None]
        idx_s[:] = idx[:, None]

    @pl.when(i == nsteps)
    def _():
        n_total = nsteps * q_ref.shape[0]
        z_mean = counts_ref[:] / n_total
        ent = jnp.sum(z_mean * jnp.log(z_mean + 1e-10), axis=1, keepdims=True)
        perp_ref[:] = jnp.exp(-ent)


def kernel(x, W):
    b, n, c = x.shape
    q = x.reshape(b * n, c)
    N = b * n
    K = W.shape[0]
    wt = W.T
    nsteps = N // N_TILE

    def cur(i):
        return jnp.minimum(i, nsteps - 1)

    def prev(i):
        return jnp.maximum(i - 1, 0)

    emb, idx, oh, perp = pl.pallas_call(
        _mq_kernel,
        grid=(nsteps + 1,),
        in_specs=[
            pl.BlockSpec((N_TILE, c), lambda i: (cur(i), 0)),
            pl.BlockSpec((K, c), lambda i: (0, 0)),
            pl.BlockSpec((c, K), lambda i: (0, 0)),
        ],
        out_specs=[
            pl.BlockSpec((N_TILE, c), lambda i: (prev(i), 0)),
            pl.BlockSpec((N_TILE, 1), lambda i: (cur(i), 0)),
            pl.BlockSpec((N_TILE, K), lambda i: (prev(i), 0)),
            pl.BlockSpec((1, 1), lambda i: (0, 0)),
        ],
        out_shape=[
            jax.ShapeDtypeStruct((N, c), jnp.float32),
            jax.ShapeDtypeStruct((N, 1), jnp.int32),
            jax.ShapeDtypeStruct((N, K), jnp.float32),
            jax.ShapeDtypeStruct((1, 1), jnp.float32),
        ],
        scratch_shapes=[
            pltpu.VMEM((N_TILE, 1), jnp.int32),
            pltpu.VMEM((1, K), jnp.float32),
        ],
    )(q, W, wt)

    embedding = emb.reshape(b, n, c)
    indices = idx.reshape(b, n)
    onehot = oh.reshape(b, n, K)
    perplexity = perp[0, 0]
    return embedding, indices, onehot, perplexity
